# baseline (speedup 1.0000x reference)
import functools

import jax
import jax.numpy as jnp
from jax import lax
from jax.experimental import pallas as pl
from jax.experimental.pallas import tpu as pltpu
from jax.experimental.pallas import tpu_sc as plsc

_NUM_CORES = 2
_NUM_SUBCORES = 16
_NUM_WORKERS = _NUM_CORES * _NUM_SUBCORES

_PAD_N = 56


@functools.partial(jax.jit, static_argnames=("batches_per_worker",))
def _sc_gather(embedding, padded_ids, batches_per_worker):
    num_padded = padded_ids.shape[0]
    d = embedding.shape[1]
    rows_per_worker = batches_per_worker * _PAD_N
    num_groups = batches_per_worker // 2
    mesh = plsc.VectorSubcoreMesh(
        core_axis_name="c",
        subcore_axis_name="s",
        num_cores=_NUM_CORES,
        num_subcores=_NUM_SUBCORES,
    )

    @functools.partial(
        pl.kernel,
        out_type=jax.ShapeDtypeStruct((num_padded, d), jnp.float32),
        mesh=mesh,
        scratch_types=[
            pltpu.VMEM((rows_per_worker,), jnp.int32),
            pltpu.VMEM((2, _PAD_N, d), jnp.float32),
            pltpu.SemaphoreType.DMA,
            pltpu.SemaphoreType.DMA,
        ],
    )
    def k(table_hbm, idx_hbm, out_hbm, idx_v, buf_v, gsem0, gsem1):
        gsems = (gsem0, gsem1)
        wid = lax.axis_index("s") * _NUM_CORES + lax.axis_index("c")
        base = wid * rows_per_worker
        pltpu.sync_copy(idx_hbm.at[pl.ds(base, rows_per_worker)], idx_v)

        def start_gather(c, b):
            idx_chunk = idx_v.at[pl.ds(c * _PAD_N, _PAD_N)]
            pltpu.async_copy(table_hbm.at[idx_chunk], buf_v.at[b], gsems[b])

        def wait_gather(c, b):
            idx_chunk = idx_v.at[pl.ds(c * _PAD_N, _PAD_N)]
            pltpu.make_async_copy(
                table_hbm.at[idx_chunk], buf_v.at[b], gsems[b]).wait()

        def scatter(c, b):
            off = pl.multiple_of(base + c * _PAD_N, 8)
            pltpu.sync_copy(buf_v.at[b], out_hbm.at[pl.ds(off, _PAD_N)])

        start_gather(0, 0)

        def body(g, _):
            c = 2 * g
            start_gather(c + 1, 1)
            wait_gather(c, 0)
            scatter(c, 0)
            start_gather(c + 2, 0)
            wait_gather(c + 1, 1)
            scatter(c + 1, 1)
            return _

        lax.fori_loop(0, num_groups - 1, body, None)

        c = batches_per_worker - 2
        start_gather(c + 1, 1)
        wait_gather(c, 0)
        scatter(c, 0)
        wait_gather(c + 1, 1)
        scatter(c + 1, 1)

    return k(embedding, padded_ids)


def kernel(relation_ids, embedding):
    b, n = relation_ids.shape
    d = embedding.shape[1]
    assert b % _NUM_WORKERS == 0 and n <= _PAD_N
    ids = relation_ids.astype(jnp.int32)
    padded = jnp.pad(ids, ((0, 0), (0, _PAD_N - n))).reshape(-1)
    out = _sc_gather(embedding.astype(jnp.float32), padded, b // _NUM_WORKERS)
    return out.reshape(b, _PAD_N, d)[:, :n, :]


# speedup vs baseline: 2.0602x; 2.0602x over previous
import functools

import jax
import jax.numpy as jnp
from jax import lax
from jax.experimental import pallas as pl
from jax.experimental.pallas import tpu as pltpu
from jax.experimental.pallas import tpu_sc as plsc

_NUM_CORES = 2
_NUM_SUBCORES = 16
_NUM_WORKERS = _NUM_CORES * _NUM_SUBCORES

_PAD_N = 56


@functools.partial(jax.jit, static_argnames=("batches_per_worker",))
def _sc_gather(embedding, padded_ids, batches_per_worker):
    num_padded = padded_ids.shape[0]
    d = embedding.shape[1]
    rows_per_worker = batches_per_worker * _PAD_N
    num_groups = batches_per_worker // 2
    mesh = plsc.VectorSubcoreMesh(
        core_axis_name="c",
        subcore_axis_name="s",
        num_cores=_NUM_CORES,
        num_subcores=_NUM_SUBCORES,
    )

    @functools.partial(
        pl.kernel,
        out_type=jax.ShapeDtypeStruct((num_padded, d), jnp.float32),
        mesh=mesh,
        scratch_types=[
            pltpu.VMEM((rows_per_worker,), jnp.int32),
            pltpu.VMEM((2, _PAD_N, d), jnp.float32),
            pltpu.SemaphoreType.DMA,
            pltpu.SemaphoreType.DMA,
        ],
    )
    def k(table_hbm, idx_hbm, out_hbm, idx_v, buf_v, gsem0, gsem1):
        gsems = (gsem0, gsem1)
        wid = lax.axis_index("s") * _NUM_CORES + lax.axis_index("c")
        base = wid * rows_per_worker
        pltpu.sync_copy(idx_hbm.at[pl.ds(base, rows_per_worker)], idx_v)

        def start_gather(c, b):
            idx_chunk = idx_v.at[pl.ds(c * _PAD_N, _PAD_N)]
            pltpu.async_copy(table_hbm.at[idx_chunk], buf_v.at[b], gsems[b])

        def wait_gather(c, b):
            idx_chunk = idx_v.at[pl.ds(c * _PAD_N, _PAD_N)]
            pltpu.make_async_copy(
                table_hbm.at[idx_chunk], buf_v.at[b], gsems[b]).wait()

        def scatter(c, b):
            off = pl.multiple_of(base + c * _PAD_N, 8)
            pltpu.sync_copy(buf_v.at[b], out_hbm.at[pl.ds(off, _PAD_N)])

        start_gather(0, 0)

        def body(g, _):
            c = 2 * g
            start_gather(c + 1, 1)
            wait_gather(c, 0)
            scatter(c, 0)
            start_gather(c + 2, 0)
            wait_gather(c + 1, 1)
            scatter(c + 1, 1)
            return _

        lax.fori_loop(0, num_groups - 1, body, None)

        c = batches_per_worker - 2
        start_gather(c + 1, 1)
        wait_gather(c, 0)
        scatter(c, 0)
        wait_gather(c + 1, 1)
        scatter(c + 1, 1)

    return k(embedding, padded_ids)


def kernel(relation_ids, embedding):
    b, n = relation_ids.shape
    d = embedding.shape[1]
    assert b % _NUM_WORKERS == 0 and n <= _PAD_N
    ids = relation_ids.astype(jnp.int32)
    padded = jnp.pad(ids, ((0, 0), (0, _PAD_N - n)), mode="wrap").reshape(-1)
    out = _sc_gather(embedding.astype(jnp.float32), padded, b // _NUM_WORKERS)
    return out.reshape(b, _PAD_N, d)[:, :n, :]
